# baseline (speedup 1.0000x reference)
import functools

import jax
import jax.numpy as jnp
import numpy as np
from jax import lax
from jax.experimental import pallas as pl
from jax.experimental.pallas import tpu as pltpu
from jax.experimental.pallas import tpu_sc as plsc

N_CAM = 6
IN_CH = 256
OUT_C = 80
FH, FW = 32, 88
HW = FH * FW
D_DEPTH = 30
NP = N_CAM * HW * D_DEPTH
NXY = 360
NCELL = NXY * NXY
NPAD_ROWS = 64
NHALF = NCELL // 2
NROWS = 65536
NR2 = 131072
NCHUNK = 5
_CW = 16

_XBOUND = (-54.0, 54.0, 0.3)
_YBOUND = (-54.0, 54.0, 0.3)
_ZBOUND = (-10.0, 10.0, 20.0)
_ROWS = (_XBOUND, _YBOUND, _ZBOUND)
_DX = np.array([r[2] for r in _ROWS], np.float32)
_BX = np.array([r[0] + r[2] / 2.0 for r in _ROWS], np.float32)


@functools.lru_cache(maxsize=1)
def _frustum():
    iH, iW = 256, 704
    ds = jnp.arange(1.0, 60.0, 2.0, dtype=jnp.float32)
    D = ds.shape[0]
    ds = jnp.broadcast_to(ds.reshape(D, 1, 1), (D, FH, FW))
    xs = jnp.broadcast_to(
        jnp.linspace(0.0, iW - 1.0, FW, dtype=jnp.float32).reshape(1, 1, FW),
        (D, FH, FW))
    ys = jnp.broadcast_to(
        jnp.linspace(0.0, iH - 1.0, FH, dtype=jnp.float32).reshape(1, FH, 1),
        (D, FH, FW))
    return jnp.stack([xs, ys, ds], -1)


def _cam_kernel(img_ref, w_ref, b_ref, depth_ref, ctx_ref):
    img = img_ref[0]
    w = w_ref[...]
    xt = lax.dot_general(img, w, (((0,), (1,)), ((), ())),
                         preferred_element_type=jnp.float32)
    xt = xt + b_ref[...]
    xd = xt[:, :D_DEPTH]
    m = jnp.max(xd, axis=1, keepdims=True)
    e = jnp.exp(xd - m)
    depth_ref[0] = e / jnp.sum(e, axis=1, keepdims=True)
    ctx_ref[0] = xt[:, D_DEPTH:D_DEPTH + OUT_C]


def _run_cam(img, w, b_row):
    return pl.pallas_call(
        _cam_kernel,
        grid=(N_CAM,),
        in_specs=[
            pl.BlockSpec((1, IN_CH, HW), lambda n: (n, 0, 0)),
            pl.BlockSpec((D_DEPTH + OUT_C, IN_CH), lambda n: (0, 0)),
            pl.BlockSpec((1, D_DEPTH + OUT_C), lambda n: (0, 0)),
        ],
        out_specs=[
            pl.BlockSpec((1, HW, D_DEPTH), lambda n: (n, 0, 0)),
            pl.BlockSpec((1, HW, OUT_C), lambda n: (n, 0, 0)),
        ],
        out_shape=[
            jax.ShapeDtypeStruct((N_CAM, HW, D_DEPTH), jnp.float32),
            jax.ShapeDtypeStruct((N_CAM, HW, OUT_C), jnp.float32),
        ],
    )(img, w, b_row)


_HB = 64
_NHB = HW // _HB


def _vals_kernel(depth_ref, mask_ref, ctx_ref, *out_refs):
    d = depth_ref[0] * mask_ref[0]
    c = ctx_ref[0]
    v = d[:, :, None] * c[:, None, :]
    for cc in range(NCHUNK):
        out_refs[cc][...] = (
            v[:, :, cc * _CW:(cc + 1) * _CW].reshape(_HB * D_DEPTH, _CW))


def _run_vals(depth, mask, ctx):
    blk = pl.BlockSpec((_HB * D_DEPTH, _CW), lambda n, hb: (n * _NHB + hb, 0))
    return pl.pallas_call(
        _vals_kernel,
        grid=(N_CAM, _NHB),
        in_specs=[
            pl.BlockSpec((1, _HB, D_DEPTH), lambda n, hb: (n, hb, 0)),
            pl.BlockSpec((1, _HB, D_DEPTH), lambda n, hb: (n, hb, 0)),
            pl.BlockSpec((1, _HB, OUT_C), lambda n, hb: (n, hb, 0)),
        ],
        out_specs=[blk] * NCHUNK,
        out_shape=[jax.ShapeDtypeStruct((NP, _CW), jnp.float32)] * NCHUNK,
    )(depth, mask, ctx)


_GROUP = 1024
_NGROUP = NP // _GROUP
_JB = _GROUP // 128
_TR0 = 4056


def _sc_scatter(*refs):
    vals_refs = refs[:NCHUNK]
    idx_lo, idx_hi, zeros_hbm, ramp_hbm, out_hbm = refs[NCHUNK:-8]
    idx_v, ramp_v, sbuf0, sbuf1, zbuf, sem0, sem1, acc_s = refs[-8:]
    core = lax.axis_index("c")
    sid = lax.axis_index("s")
    bufs = (sbuf0, sbuf1)
    sems = (sem0, sem1)

    pltpu.sync_copy(zeros_hbm, zbuf)
    pltpu.sync_copy(ramp_hbm.at[sid], ramp_v)

    n_g = jnp.where(sid == 15, 30, 31)
    g0 = sid * 31

    def _rowcopy(fn):
        for z in range(30):
            fn(z, 128)

        @pl.when(sid < 15)
        def _p1():
            fn(30, 128)
            fn(31, 88)

        @pl.when(sid == 15)
        def _p2():
            fn(30, 120)

    def _do_chunk(vals_hbm, idx_hbm, cc, out_base):
        def _zero(z, nr):
            pltpu.sync_copy(zbuf.at[pl.ds(0, nr)],
                            acc_s.at[ramp_v.at[z, pl.ds(0, nr)]])
        _rowcopy(_zero)
        plsc.subcore_barrier()

        def _group(g, carry):
            ga = g0 + g
            pltpu.sync_copy(idx_hbm.at[pl.ds(ga * _JB, _JB)], idx_v)
            cps = [None, None]
            cps[0] = pltpu.async_copy(
                vals_hbm.at[pl.ds(ga * _GROUP, 128)], bufs[0], sems[0])
            for j in range(_JB):
                cur = j % 2
                cps[cur].wait()
                if j + 1 < _JB:
                    cps[1 - cur] = pltpu.async_copy(
                        vals_hbm.at[pl.ds(ga * _GROUP + (j + 1) * 128, 128)],
                        bufs[1 - cur], sems[1 - cur])
                pltpu.sync_copy(bufs[cur], acc_s.at[idx_v.at[j]], add=True)
            return carry
        lax.fori_loop(0, n_g, _group, 0)
        plsc.subcore_barrier()

        def _wb(z, nr):
            pltpu.sync_copy(acc_s.at[ramp_v.at[z, pl.ds(0, nr)]],
                            sbuf0.at[pl.ds(0, nr)])
            pltpu.sync_copy(
                sbuf0.at[pl.ds(0, nr)],
                out_hbm.at[cc, pl.ds(out_base + sid * _TR0 + z * 128, nr)])
        _rowcopy(_wb)

    for k in range(NCHUNK):
        @pl.when(core == 0)
        def _c0(k=k):
            _do_chunk(vals_refs[k], idx_lo, k, 0)

        @pl.when(core == 1)
        def _c1(k=k):
            _do_chunk(vals_refs[k], idx_hi, k, NHALF)


def _run_scatter(vals_list, idx_lo, idx_hi, zeros, ramp):
    mesh = plsc.VectorSubcoreMesh(core_axis_name="c", subcore_axis_name="s",
                                  num_cores=2, num_subcores=16)
    return pl.kernel(
        _sc_scatter,
        out_type=jax.ShapeDtypeStruct((NCHUNK, NR2, _CW), jnp.float32),
        mesh=mesh,
        scratch_types=[
            pltpu.MemorySpace.VMEM((_JB, 128), jnp.int32),
            pltpu.MemorySpace.VMEM((32, 128), jnp.int32),
            pltpu.MemorySpace.VMEM((128, _CW), jnp.float32),
            pltpu.MemorySpace.VMEM((128, _CW), jnp.float32),
            pltpu.MemorySpace.VMEM((128, _CW), jnp.float32),
            pltpu.SemaphoreType.DMA,
            pltpu.SemaphoreType.DMA,
            pltpu.MemorySpace.VMEM_SHARED((NROWS, _CW), jnp.float32),
        ],
    )(*vals_list, idx_lo, idx_hi, zeros, ramp)


_BR = 4096


def _tr_kernel(acc_ref, out_ref):
    out_ref[0] = acc_ref[0].T


def _run_transpose(acc):
    return pl.pallas_call(
        _tr_kernel,
        grid=(NCHUNK, NR2 // _BR),
        in_specs=[pl.BlockSpec((1, _BR, _CW), lambda cc, rb: (cc, rb, 0))],
        out_specs=pl.BlockSpec((1, _CW, _BR), lambda cc, rb: (0, cc, rb)),
        out_shape=jax.ShapeDtypeStruct((1, OUT_C, NR2), jnp.float32),
    )(acc)


def _point_index(img_aug_matrix, lidar2image):
    pts = jnp.broadcast_to(_frustum()[None, None],
                           (1, N_CAM, D_DEPTH, FH, FW, 3))
    pts = jnp.concatenate([pts, jnp.ones_like(pts[..., :1])], -1)
    inv_aug = jnp.linalg.inv(img_aug_matrix)
    pts = jnp.einsum('bnij,bndhwj->bndhwi', inv_aug, pts)
    pts = jnp.concatenate([pts[..., :2] * pts[..., 2:3], pts[..., 2:3],
                           jnp.ones_like(pts[..., 2:3])], -1)
    inv_l2i = jnp.linalg.inv(lidar2image)
    geom = jnp.einsum('bnij,bndhwj->bndhwi', inv_l2i, pts)[..., :3]
    bx = jnp.asarray(_BX)
    dx = jnp.asarray(_DX)
    g = ((geom - (bx - dx / 2.0)) / dx).astype(jnp.int32).reshape(NP, 3)
    kept = ((g[:, 0] >= 0) & (g[:, 0] < NXY) & (g[:, 1] >= 0)
            & (g[:, 1] < NXY) & (g[:, 2] >= 0) & (g[:, 2] < 1))
    gx = jnp.clip(g[:, 0], 0, NXY - 1)
    gy = jnp.clip(g[:, 1], 0, NXY - 1)
    pad = NHALF + (jnp.arange(NP, dtype=jnp.int32) % HW) % NPAD_ROWS
    flat = jnp.where(kept, gy * NXY + gx, -1)
    row_lo = jnp.where(kept & (flat < NHALF), flat, pad)
    row_hi = jnp.where(kept & (flat >= NHALF), flat - NHALF, pad)

    def _reorder(a):
        return a.reshape(N_CAM, D_DEPTH, HW).transpose(0, 2, 1)

    return (_reorder(row_lo), _reorder(row_hi),
            _reorder(kept.astype(jnp.float32)))


def kernel(image_fpn, img_aug_matrix, lidar2image, W_depth, b_depth):
    img = image_fpn.reshape(N_CAM, IN_CH, HW)
    b_row = b_depth.reshape(1, D_DEPTH + OUT_C)

    row_lo, row_hi, mask3 = _point_index(img_aug_matrix, lidar2image)
    depth, ctx = _run_cam(img, W_depth, b_row)
    vals_list = _run_vals(depth, mask3, ctx)
    idx_lo = row_lo.reshape(NP // 128, 128)
    idx_hi = row_hi.reshape(NP // 128, 128)
    zeros = jnp.zeros((128, _CW), jnp.float32)
    ramp = (jnp.arange(16, dtype=jnp.int32)[:, None, None] * _TR0
            + jnp.arange(32, dtype=jnp.int32)[None, :, None] * 128
            + jnp.arange(128, dtype=jnp.int32)[None, None, :])
    acc = _run_scatter(vals_list, idx_lo, idx_hi, zeros, ramp)
    out3 = _run_transpose(acc)
    return out3[:, :, :NCELL].reshape(1, OUT_C, NXY, NXY)
